# baseline (speedup 1.0000x reference)
import functools

import jax
import jax.numpy as jnp
from jax import lax
from jax.experimental import pallas as pl
from jax.experimental.pallas import tpu as pltpu
from jax.experimental.pallas import tpu_sc as plsc

B = 16384
N_ROWS = 100000
GMF_DIM = 64
MLP_DIM = 32
CH = 128
LN = 16
BLKL = 9088
NMAIN = 99968


GMF_SPLIT = 50048
GMF_B2 = 2176
GMF_NBLK = 23
MLP_SPLIT = 25088
MLP_B2 = 3584
MLP_NBLK = 7


def _transpose_fold(tbl_t, d, fold):
    if fold == 2:
        split, b2, nblk = GMF_SPLIT, GMF_B2, GMF_NBLK
    else:
        split, b2, nblk = MLP_SPLIT, MLP_B2, MLP_NBLK

    def body(*refs):
        out_ref = refs[-1]
        for f in range(fold):
            out_ref[:, f * d:(f + 1) * d] = refs[f][:].T

    in_specs = [
        pl.BlockSpec((d, b2), functools.partial(
            lambda f, i: (0, f * nblk + i), f))
        for f in range(fold)
    ]
    return pl.pallas_call(
        body,
        grid=(nblk,),
        in_specs=in_specs,
        out_specs=pl.BlockSpec((b2, 128), lambda i: (i, 0)),
        out_shape=jax.ShapeDtypeStruct((split, 128), jnp.float32),
    )(*([tbl_t] * fold))


def _make_gather_kernel(b_per_w):
    mesh = plsc.VectorSubcoreMesh(core_axis_name="c", subcore_axis_name="s")
    info = plsc.get_sparse_core_info()
    nc = info.num_cores

    @functools.partial(
        pl.kernel,
        mesh=mesh,
        out_type=[
            jax.ShapeDtypeStruct((B, 128), jnp.float32),
            jax.ShapeDtypeStruct((B, 128), jnp.float32),
            jax.ShapeDtypeStruct((B, 128), jnp.float32),
            jax.ShapeDtypeStruct((B, 128), jnp.float32),
        ],
        scratch_types=[
            pltpu.VMEM((b_per_w,), jnp.int32),
            pltpu.VMEM((b_per_w,), jnp.int32),
            pltpu.VMEM((CH, 128), jnp.float32),
            pltpu.VMEM((CH, 128), jnp.float32),
            pltpu.VMEM((CH, 128), jnp.float32),
            pltpu.VMEM((CH, 128), jnp.float32),
            pltpu.SemaphoreType.DMA,
        ],
    )
    def gather_kernel(uidx_hbm, iidx_hbm, gmf_user_hbm, gmf_item_hbm,
                      mlp_user_hbm, mlp_item_hbm,
                      gu_out, gi_out, mu_out, mi_out,
                      uidx_v, iidx_v, gu_v, gi_v, mu_v, mi_v, sem):
        wid = lax.axis_index("s") * nc + lax.axis_index("c")
        base = wid * b_per_w
        pltpu.sync_copy(uidx_hbm.at[pl.ds(base, b_per_w)], uidx_v)
        pltpu.sync_copy(iidx_hbm.at[pl.ds(base, b_per_w)], iidx_v)

        def chunk(c, _):
            def fire(g, _):
                uvec = uidx_v[pl.ds(c * CH + g * LN, LN)]
                ivec = iidx_v[pl.ds(c * CH + g * LN, LN)]
                ugvec = lax.rem(uvec, GMF_SPLIT)
                igvec = lax.rem(ivec, GMF_SPLIT)
                umvec = lax.rem(uvec, MLP_SPLIT)
                imvec = lax.rem(ivec, MLP_SPLIT)
                for l in range(LN):
                    j = g * LN + l
                    pltpu.async_copy(
                        gmf_user_hbm.at[pl.ds(ugvec[l], 1)],
                        gu_v.at[pl.ds(j, 1)], sem)
                    pltpu.async_copy(
                        gmf_item_hbm.at[pl.ds(igvec[l], 1)],
                        gi_v.at[pl.ds(j, 1)], sem)
                    pltpu.async_copy(
                        mlp_user_hbm.at[pl.ds(umvec[l], 1)],
                        mu_v.at[pl.ds(j, 1)], sem)
                    pltpu.async_copy(
                        mlp_item_hbm.at[pl.ds(imvec[l], 1)],
                        mi_v.at[pl.ds(j, 1)], sem)
                return 0

            lax.fori_loop(0, CH // LN, fire, 0)

            def drain(j, _):
                for ref in (gu_v, gi_v, mu_v, mi_v):
                    pltpu.make_async_copy(
                        gmf_user_hbm.at[pl.ds(0, 1)], ref.at[pl.ds(j, 1)],
                        sem).wait()
                return 0

            lax.fori_loop(0, CH, drain, 0)

            pltpu.sync_copy(gu_v, gu_out.at[pl.ds(base + c * CH, CH)])
            pltpu.sync_copy(gi_v, gi_out.at[pl.ds(base + c * CH, CH)])
            pltpu.sync_copy(mu_v, mu_out.at[pl.ds(base + c * CH, CH)])
            pltpu.sync_copy(mi_v, mi_out.at[pl.ds(base + c * CH, CH)])
            return 0

        lax.fori_loop(0, b_per_w // CH, chunk, 0)

    return gather_kernel


def _dense_body(gu2, gi2, mu2, mi2, sgu, sgi, m0u, m1u, m0i, m1i,
                w1a, w1b, b1, w2, b2, w3, b3, wfg, wfh, bf, out):
    su = sgu[:]
    si = sgi[:]
    gu = gu2[:, :GMF_DIM] * (1.0 - su) + gu2[:, GMF_DIM:] * su
    gi = gi2[:, :GMF_DIM] * (1.0 - si) + gi2[:, GMF_DIM:] * si

    def quarter(x2, m0, m1):
        a = x2[:, 0:32] * (1.0 - m0) + x2[:, 32:64] * m0
        b = x2[:, 64:96] * (1.0 - m0) + x2[:, 96:128] * m0
        return a * (1.0 - m1) + b * m1

    mu = quarter(mu2[:], m0u[:], m1u[:])
    mi = quarter(mi2[:], m0i[:], m1i[:])

    h = jnp.dot(mu, w1a[:], preferred_element_type=jnp.float32)
    h = h + jnp.dot(mi, w1b[:], preferred_element_type=jnp.float32)
    h = jnp.maximum(h + b1[:], 0.0)
    h = jnp.maximum(
        jnp.dot(h, w2[:], preferred_element_type=jnp.float32) + b2[:], 0.0)
    h = jnp.maximum(
        jnp.dot(h, w3[:], preferred_element_type=jnp.float32) + b3[:], 0.0)
    g = gu * gi
    s = jnp.dot(g, wfg[:], preferred_element_type=jnp.float32)
    s = s + jnp.dot(h, wfh[:], preferred_element_type=jnp.float32)
    out[:] = jax.nn.sigmoid(s + bf[:])


def kernel(user_indices, item_indices, gmf_user, gmf_item, mlp_user,
           mlp_item, W1, b1, W2, b2, W3, b3, Wf, bf):
    user_indices = user_indices.astype(jnp.int32)
    item_indices = item_indices.astype(jnp.int32)

    gmf_u2 = _transpose_fold(gmf_user.T, GMF_DIM, 2)
    gmf_i2 = _transpose_fold(gmf_item.T, GMF_DIM, 2)
    mlp_u2 = _transpose_fold(mlp_user.T, MLP_DIM, 4)
    mlp_i2 = _transpose_fold(mlp_item.T, MLP_DIM, 4)

    nw = 32
    b_per_w = B // nw
    gu2, gi2, mu2, mi2 = _make_gather_kernel(b_per_w)(
        user_indices, item_indices, gmf_u2, gmf_i2, mlp_u2, mlp_i2)

    sgu = (user_indices >= GMF_SPLIT).astype(jnp.float32).reshape(B, 1)
    sgi = (item_indices >= GMF_SPLIT).astype(jnp.float32).reshape(B, 1)

    def qidx(idx):
        return ((idx >= MLP_SPLIT).astype(jnp.int32)
                + (idx >= 2 * MLP_SPLIT).astype(jnp.int32)
                + (idx >= 3 * MLP_SPLIT).astype(jnp.int32))

    qu = qidx(user_indices)
    qi = qidx(item_indices)
    m0u = (qu & 1).astype(jnp.float32).reshape(B, 1)
    m1u = (qu >> 1).astype(jnp.float32).reshape(B, 1)
    m0i = (qi & 1).astype(jnp.float32).reshape(B, 1)
    m1i = (qi >> 1).astype(jnp.float32).reshape(B, 1)

    blk = 2048
    grid = B // blk
    w1a = W1[:MLP_DIM]
    w1b = W1[MLP_DIM:]
    wfg = Wf[:GMF_DIM]
    wfh = Wf[GMF_DIM:]
    rep = lambda shape: pl.BlockSpec(shape, lambda i: (0, 0))
    blkspec = lambda w: pl.BlockSpec((blk, w), lambda i: (i, 0))
    out = pl.pallas_call(
        _dense_body,
        grid=(grid,),
        in_specs=[
            blkspec(128), blkspec(128), blkspec(128), blkspec(128),
            blkspec(1), blkspec(1), blkspec(1), blkspec(1), blkspec(1),
            blkspec(1),
            rep((MLP_DIM, 128)),
            rep((MLP_DIM, 128)),
            rep((1, 128)),
            rep((128, 64)),
            rep((1, 64)),
            rep((64, 32)),
            rep((1, 32)),
            rep((GMF_DIM, 1)),
            rep((32, 1)),
            rep((1, 1)),
        ],
        out_specs=pl.BlockSpec((blk, 1), lambda i: (i, 0)),
        out_shape=jax.ShapeDtypeStruct((B, 1), jnp.float32),
    )(gu2, gi2, mu2, mi2, sgu, sgi, m0u, m1u, m0i, m1i,
      w1a, w1b, b1.reshape(1, -1), W2, b2.reshape(1, -1),
      W3, b3.reshape(1, -1), wfg, wfh, bf.reshape(1, 1))
    return out[:, 0]


# speedup vs baseline: 1.2613x; 1.2613x over previous
import functools

import jax
import jax.numpy as jnp
from jax import lax
from jax.experimental import pallas as pl
from jax.experimental.pallas import tpu as pltpu
from jax.experimental.pallas import tpu_sc as plsc

B = 16384
GMF_DIM = 64
MLP_DIM = 32
CH = 128
LN = 16
NW = 32


def _make_gather_kernel(d):
    b_per_w = B // NW
    mesh = plsc.VectorSubcoreMesh(core_axis_name="c", subcore_axis_name="s")
    info = plsc.get_sparse_core_info()
    nc = info.num_cores

    @functools.partial(
        pl.kernel,
        mesh=mesh,
        out_type=[jax.ShapeDtypeStruct((B, d), jnp.float32)],
        scratch_types=[
            pltpu.VMEM((b_per_w,), jnp.int32),
            pltpu.VMEM((CH, d), jnp.float32),
            pltpu.SemaphoreType.DMA,
        ],
    )
    def gather_kernel(idx_hbm, tbl_hbm, rows_out, idx_v, row_v, sem):
        wid = lax.axis_index("s") * nc + lax.axis_index("c")
        base = wid * b_per_w
        pltpu.sync_copy(idx_hbm.at[pl.ds(base, b_per_w)], idx_v)

        def chunk(c, _):
            def fire(g, _):
                vec = idx_v[pl.ds(c * CH + g * LN, LN)]
                for l in range(LN):
                    pltpu.async_copy(
                        tbl_hbm.at[pl.ds(vec[l], 1)],
                        row_v.at[pl.ds(g * LN + l, 1)], sem)
                return 0

            lax.fori_loop(0, CH // LN, fire, 0)

            def drain(j, _):
                pltpu.make_async_copy(
                    tbl_hbm.at[pl.ds(0, 1)], row_v.at[pl.ds(j, 1)],
                    sem).wait()
                return 0

            lax.fori_loop(0, CH, drain, 0)
            pltpu.sync_copy(row_v, rows_out.at[pl.ds(base + c * CH, CH)])
            return 0

        lax.fori_loop(0, b_per_w // CH, chunk, 0)

    return gather_kernel


def _dense_body(gu, gi, mu, mi, w1a, w1b, b1, w2, b2, w3, b3, wfg, wfh, bf,
                out):
    h = jnp.dot(mu[:], w1a[:], preferred_element_type=jnp.float32)
    h = h + jnp.dot(mi[:], w1b[:], preferred_element_type=jnp.float32)
    h = jnp.maximum(h + b1[:], 0.0)
    h = jnp.maximum(
        jnp.dot(h, w2[:], preferred_element_type=jnp.float32) + b2[:], 0.0)
    h = jnp.maximum(
        jnp.dot(h, w3[:], preferred_element_type=jnp.float32) + b3[:], 0.0)
    g = gu[:] * gi[:]
    s = jnp.dot(g, wfg[:], preferred_element_type=jnp.float32)
    s = s + jnp.dot(h, wfh[:], preferred_element_type=jnp.float32)
    out[:] = jax.nn.sigmoid(s + bf[:])


def kernel(user_indices, item_indices, gmf_user, gmf_item, mlp_user,
           mlp_item, W1, b1, W2, b2, W3, b3, Wf, bf):
    user_indices = user_indices.astype(jnp.int32)
    item_indices = item_indices.astype(jnp.int32)

    gather64 = _make_gather_kernel(GMF_DIM)
    gather32 = _make_gather_kernel(MLP_DIM)
    (gu,) = gather64(user_indices, gmf_user)
    (gi,) = gather64(item_indices, gmf_item)
    (mu,) = gather32(user_indices, mlp_user)
    (mi,) = gather32(item_indices, mlp_item)

    blk = 2048
    grid = B // blk
    w1a = W1[:MLP_DIM]
    w1b = W1[MLP_DIM:]
    wfg = Wf[:GMF_DIM]
    wfh = Wf[GMF_DIM:]
    rep = lambda shape: pl.BlockSpec(shape, lambda i: (0, 0))
    out = pl.pallas_call(
        _dense_body,
        grid=(grid,),
        in_specs=[
            pl.BlockSpec((blk, GMF_DIM), lambda i: (i, 0)),
            pl.BlockSpec((blk, GMF_DIM), lambda i: (i, 0)),
            pl.BlockSpec((blk, MLP_DIM), lambda i: (i, 0)),
            pl.BlockSpec((blk, MLP_DIM), lambda i: (i, 0)),
            rep((MLP_DIM, 128)),
            rep((MLP_DIM, 128)),
            rep((1, 128)),
            rep((128, 64)),
            rep((1, 64)),
            rep((64, 32)),
            rep((1, 32)),
            rep((GMF_DIM, 1)),
            rep((32, 1)),
            rep((1, 1)),
        ],
        out_specs=pl.BlockSpec((blk, 1), lambda i: (i, 0)),
        out_shape=jax.ShapeDtypeStruct((B, 1), jnp.float32),
    )(gu, gi, mu, mi, w1a, w1b, b1.reshape(1, -1), W2, b2.reshape(1, -1),
      W3, b3.reshape(1, -1), wfg, wfh, bf.reshape(1, 1))
    return out.reshape(B)


# speedup vs baseline: 1.2811x; 1.0156x over previous
import functools

import jax
import jax.numpy as jnp
from jax import lax
from jax.experimental import pallas as pl
from jax.experimental.pallas import tpu as pltpu
from jax.experimental.pallas import tpu_sc as plsc

B = 16384
GMF_DIM = 64
MLP_DIM = 32
CH = 128
LN = 16
NW = 32


def _make_gather_kernel(d):
    b_per_w = B // NW
    mesh = plsc.VectorSubcoreMesh(core_axis_name="c", subcore_axis_name="s")
    info = plsc.get_sparse_core_info()
    nc = info.num_cores

    @functools.partial(
        pl.kernel,
        mesh=mesh,
        out_type=[jax.ShapeDtypeStruct((B, d), jnp.float32)],
        scratch_types=[
            pltpu.VMEM((b_per_w,), jnp.int32),
            pltpu.VMEM((CH, d), jnp.float32),
            pltpu.SemaphoreType.DMA,
        ],
    )
    def gather_kernel(idx_hbm, tbl_hbm, rows_out, idx_v, row_v, sem):
        wid = lax.axis_index("s") * nc + lax.axis_index("c")
        base = wid * b_per_w
        pltpu.sync_copy(idx_hbm.at[pl.ds(base, b_per_w)], idx_v)

        def chunk(c, _):
            def fire(g, _):
                vec = idx_v[pl.ds(c * CH + g * LN, LN)]
                for l in range(LN):
                    pltpu.async_copy(
                        tbl_hbm.at[pl.ds(vec[l], 1)],
                        row_v.at[pl.ds(g * LN + l, 1)], sem)
                return 0

            lax.fori_loop(0, CH // LN, fire, 0)

            def drain(j, _):
                pltpu.make_async_copy(
                    tbl_hbm.at[pl.ds(0, 1)], row_v.at[pl.ds(j, 1)],
                    sem).wait()
                return 0

            lax.fori_loop(0, CH, drain, 0)
            pltpu.sync_copy(row_v, rows_out.at[pl.ds(base + c * CH, CH)])
            return 0

        lax.fori_loop(0, b_per_w // CH, chunk, 0)

    return gather_kernel


def _dense_body(gu, gi, mu, mi, w1a, w1b, b1, w2, b2, w3, b3, wfg, wfh, bf,
                out):
    h = jnp.dot(mu[:], w1a[:], preferred_element_type=jnp.float32)
    h = h + jnp.dot(mi[:], w1b[:], preferred_element_type=jnp.float32)
    h = jnp.maximum(h + b1[:], 0.0)
    h = jnp.maximum(
        jnp.dot(h, w2[:], preferred_element_type=jnp.float32) + b2[:], 0.0)
    h = jnp.maximum(
        jnp.dot(h, w3[:], preferred_element_type=jnp.float32) + b3[:], 0.0)
    g = gu[:] * gi[:]
    s = jnp.dot(g, wfg[:], preferred_element_type=jnp.float32)
    s = s + jnp.dot(h, wfh[:], preferred_element_type=jnp.float32)
    out[:] = jax.nn.sigmoid(s + bf[:]).reshape(out.shape)


def kernel(user_indices, item_indices, gmf_user, gmf_item, mlp_user,
           mlp_item, W1, b1, W2, b2, W3, b3, Wf, bf):
    user_indices = user_indices.astype(jnp.int32)
    item_indices = item_indices.astype(jnp.int32)

    gather64 = _make_gather_kernel(GMF_DIM)
    gather32 = _make_gather_kernel(MLP_DIM)
    (gu,) = gather64(user_indices, gmf_user)
    (gi,) = gather64(item_indices, gmf_item)
    (mu,) = gather32(user_indices, mlp_user)
    (mi,) = gather32(item_indices, mlp_item)

    blk = 2048
    grid = B // blk
    w1a = W1[:MLP_DIM]
    w1b = W1[MLP_DIM:]
    wfg = Wf[:GMF_DIM]
    wfh = Wf[GMF_DIM:]
    rep = lambda shape: pl.BlockSpec(shape, lambda i: (0, 0))
    out = pl.pallas_call(
        _dense_body,
        grid=(grid,),
        in_specs=[
            pl.BlockSpec((blk, GMF_DIM), lambda i: (i, 0)),
            pl.BlockSpec((blk, GMF_DIM), lambda i: (i, 0)),
            pl.BlockSpec((blk, MLP_DIM), lambda i: (i, 0)),
            pl.BlockSpec((blk, MLP_DIM), lambda i: (i, 0)),
            rep((MLP_DIM, 128)),
            rep((MLP_DIM, 128)),
            rep((1, 128)),
            rep((128, 64)),
            rep((1, 64)),
            rep((64, 32)),
            rep((1, 32)),
            rep((GMF_DIM, 1)),
            rep((32, 1)),
            rep((1, 1)),
        ],
        out_specs=pl.BlockSpec((blk,), lambda i: (i,)),
        out_shape=jax.ShapeDtypeStruct((B,), jnp.float32),
    )(gu, gi, mu, mi, w1a, w1b, b1.reshape(1, -1), W2, b2.reshape(1, -1),
      W3, b3.reshape(1, -1), wfg, wfh, bf.reshape(1, 1))
    return out


# speedup vs baseline: 1.3425x; 1.0479x over previous
import functools

import jax
import jax.numpy as jnp
from jax import lax
from jax.experimental import pallas as pl
from jax.experimental.pallas import tpu as pltpu
from jax.experimental.pallas import tpu_sc as plsc

B = 16384
GMF_DIM = 64
MLP_DIM = 32
CH = 128
LN = 16
NW = 32
MLP_SPLIT = 25088
MLP_B2 = 3584
MLP_NBLK = 7


def _transpose_fold_mlp(tbl_t):
    def body(*refs):
        out_ref = refs[-1]
        for f in range(4):
            out_ref[:, f * MLP_DIM:(f + 1) * MLP_DIM] = refs[f][:].T

    in_specs = [
        pl.BlockSpec((MLP_DIM, MLP_B2), functools.partial(
            lambda f, i: (0, f * MLP_NBLK + i), f))
        for f in range(4)
    ]
    return pl.pallas_call(
        body,
        grid=(MLP_NBLK,),
        in_specs=in_specs,
        out_specs=pl.BlockSpec((MLP_B2, 128), lambda i: (i, 0)),
        out_shape=jax.ShapeDtypeStruct((MLP_SPLIT, 128), jnp.float32),
    )(*([tbl_t] * 4))


def _make_gather_kernel_folded():
    d = MLP_DIM
    b_per_w = B // NW
    mesh = plsc.VectorSubcoreMesh(core_axis_name="c", subcore_axis_name="s")
    info = plsc.get_sparse_core_info()
    nc = info.num_cores

    @functools.partial(
        pl.kernel,
        mesh=mesh,
        out_type=[jax.ShapeDtypeStruct((B, d), jnp.float32)],
        scratch_types=[
            pltpu.VMEM((b_per_w,), jnp.int32),
            pltpu.VMEM((CH, 128), jnp.float32),
            pltpu.VMEM((CH, d), jnp.float32),
            pltpu.SemaphoreType.DMA,
        ],
    )
    def gather_kernel(idx_hbm, tbl_hbm, rows_out, idx_v, line_v, row_v, sem):
        wid = lax.axis_index("s") * nc + lax.axis_index("c")
        base = wid * b_per_w
        pltpu.sync_copy(idx_hbm.at[pl.ds(base, b_per_w)], idx_v)

        def chunk(c, _):
            def fire(g, _):
                vec = idx_v[pl.ds(c * CH + g * LN, LN)]
                kvec = lax.rem(vec, MLP_SPLIT)
                for l in range(LN):
                    pltpu.async_copy(
                        tbl_hbm.at[pl.ds(kvec[l], 1)],
                        line_v.at[pl.ds(g * LN + l, 1)], sem)
                return 0

            lax.fori_loop(0, CH // LN, fire, 0)

            def drain(j, _):
                pltpu.make_async_copy(
                    tbl_hbm.at[pl.ds(0, 1)], line_v.at[pl.ds(j, 1)],
                    sem).wait()
                return 0

            lax.fori_loop(0, CH, drain, 0)

            def extract(g, _):
                vec = idx_v[pl.ds(c * CH + g * LN, LN)]
                ovec = lax.div(vec, MLP_SPLIT) * d
                for l in range(LN):
                    j = g * LN + l
                    o = ovec[l]
                    row_v[j, pl.ds(0, 16)] = line_v[j, pl.ds(o, 16)]
                    row_v[j, pl.ds(16, 16)] = line_v[j, pl.ds(o + 16, 16)]
                return 0

            lax.fori_loop(0, CH // LN, extract, 0)
            pltpu.sync_copy(row_v, rows_out.at[pl.ds(base + c * CH, CH)])
            return 0

        lax.fori_loop(0, b_per_w // CH, chunk, 0)

    return gather_kernel


def _make_gather_kernel(d):
    b_per_w = B // NW
    mesh = plsc.VectorSubcoreMesh(core_axis_name="c", subcore_axis_name="s")
    info = plsc.get_sparse_core_info()
    nc = info.num_cores

    @functools.partial(
        pl.kernel,
        mesh=mesh,
        out_type=[jax.ShapeDtypeStruct((B, d), jnp.float32)],
        scratch_types=[
            pltpu.VMEM((b_per_w,), jnp.int32),
            pltpu.VMEM((CH, d), jnp.float32),
            pltpu.SemaphoreType.DMA,
        ],
    )
    def gather_kernel(idx_hbm, tbl_hbm, rows_out, idx_v, row_v, sem):
        wid = lax.axis_index("s") * nc + lax.axis_index("c")
        base = wid * b_per_w
        pltpu.sync_copy(idx_hbm.at[pl.ds(base, b_per_w)], idx_v)

        def chunk(c, _):
            def fire(g, _):
                vec = idx_v[pl.ds(c * CH + g * LN, LN)]
                for l in range(LN):
                    pltpu.async_copy(
                        tbl_hbm.at[pl.ds(vec[l], 1)],
                        row_v.at[pl.ds(g * LN + l, 1)], sem)
                return 0

            lax.fori_loop(0, CH // LN, fire, 0)

            def drain(j, _):
                pltpu.make_async_copy(
                    tbl_hbm.at[pl.ds(0, 1)], row_v.at[pl.ds(j, 1)],
                    sem).wait()
                return 0

            lax.fori_loop(0, CH, drain, 0)
            pltpu.sync_copy(row_v, rows_out.at[pl.ds(base + c * CH, CH)])
            return 0

        lax.fori_loop(0, b_per_w // CH, chunk, 0)

    return gather_kernel


def _dense_body(gu, gi, mu, mi, w1a, w1b, b1, w2, b2, w3, b3, wfg, wfh, bf,
                out):
    h = jnp.dot(mu[:], w1a[:], preferred_element_type=jnp.float32)
    h = h + jnp.dot(mi[:], w1b[:], preferred_element_type=jnp.float32)
    h = jnp.maximum(h + b1[:], 0.0)
    h = jnp.maximum(
        jnp.dot(h, w2[:], preferred_element_type=jnp.float32) + b2[:], 0.0)
    h = jnp.maximum(
        jnp.dot(h, w3[:], preferred_element_type=jnp.float32) + b3[:], 0.0)
    g = gu[:] * gi[:]
    s = jnp.dot(g, wfg[:], preferred_element_type=jnp.float32)
    s = s + jnp.dot(h, wfh[:], preferred_element_type=jnp.float32)
    out[:] = jax.nn.sigmoid(s + bf[:]).reshape(out.shape)


def kernel(user_indices, item_indices, gmf_user, gmf_item, mlp_user,
           mlp_item, W1, b1, W2, b2, W3, b3, Wf, bf):
    user_indices = user_indices.astype(jnp.int32)
    item_indices = item_indices.astype(jnp.int32)

    gather64 = _make_gather_kernel(GMF_DIM)
    gather32 = _make_gather_kernel_folded()
    mlp_u2 = _transpose_fold_mlp(mlp_user.T)
    mlp_i2 = _transpose_fold_mlp(mlp_item.T)
    (gu,) = gather64(user_indices, gmf_user)
    (gi,) = gather64(item_indices, gmf_item)
    (mu,) = gather32(user_indices, mlp_u2)
    (mi,) = gather32(item_indices, mlp_i2)

    blk = 2048
    grid = B // blk
    w1a = W1[:MLP_DIM]
    w1b = W1[MLP_DIM:]
    wfg = Wf[:GMF_DIM]
    wfh = Wf[GMF_DIM:]
    rep = lambda shape: pl.BlockSpec(shape, lambda i: (0, 0))
    out = pl.pallas_call(
        _dense_body,
        grid=(grid,),
        in_specs=[
            pl.BlockSpec((blk, GMF_DIM), lambda i: (i, 0)),
            pl.BlockSpec((blk, GMF_DIM), lambda i: (i, 0)),
            pl.BlockSpec((blk, MLP_DIM), lambda i: (i, 0)),
            pl.BlockSpec((blk, MLP_DIM), lambda i: (i, 0)),
            rep((MLP_DIM, 128)),
            rep((MLP_DIM, 128)),
            rep((1, 128)),
            rep((128, 64)),
            rep((1, 64)),
            rep((64, 32)),
            rep((1, 32)),
            rep((GMF_DIM, 1)),
            rep((32, 1)),
            rep((1, 1)),
        ],
        out_specs=pl.BlockSpec((blk,), lambda i: (i,)),
        out_shape=jax.ShapeDtypeStruct((B,), jnp.float32),
    )(gu, gi, mu, mi, w1a, w1b, b1.reshape(1, -1), W2, b2.reshape(1, -1),
      W3, b3.reshape(1, -1), wfg, wfh, bf.reshape(1, 1))
    return out
